# baseline (speedup 1.0000x reference)
import functools

import jax
import jax.numpy as jnp
from jax import lax
from jax.experimental import pallas as pl
from jax.experimental.pallas import tpu as pltpu
from jax.experimental.pallas import tpu_sc as plsc

_K = 32
_BLOCK_R = 128
_TOP = 5
_N = 10000
_CAND_W = _TOP * 128 + 16
_R_PAD = 10240
_RPW = _R_PAD // 32
_BATCH = 16


def _cand_body(a_ref, n_ref, c_ref):
    rows = a_ref.shape[0]
    cols = a_ref.shape[1]
    full = cols // 128
    tail = cols - full * 128
    neg1 = jnp.int32(-1)

    for strip in range(rows // 8):
        r0 = strip * 8

        def step(j, ms):
            a = a_ref[r0:r0 + 8, pl.ds(j * 128, 128)]
            n = n_ref[r0:r0 + 8, pl.ds(j * 128, 128)]
            x = jax.lax.bitcast_convert_type(
                jnp.maximum(a, 0.0) + n, jnp.int32)
            out = []
            for m in ms:
                t = jnp.maximum(m, x)
                x = jnp.minimum(m, x)
                out.append(t)
            return tuple(out)

        init = tuple(jnp.full((8, 128), neg1) for _ in range(_TOP))
        ms = jax.lax.fori_loop(0, full, step, init)
        for i, m in enumerate(ms):
            c_ref[r0:r0 + 8, i * 128:(i + 1) * 128] = m
        at = a_ref[r0:r0 + 8, full * 128:cols]
        nt = n_ref[r0:r0 + 8, full * 128:cols]
        vt = jax.lax.bitcast_convert_type(jnp.maximum(at, 0.0) + nt,
                                          jnp.int32)
        c_ref[r0:r0 + 8, _TOP * 128:_TOP * 128 + tail] = vt


def _sc_select_body(ct_hbm, v_hbm, bbuf, vbuf, sem):
    wid = lax.axis_index("s") * 2 + lax.axis_index("c")
    gbase = wid * (_RPW // 16)
    one = jnp.int32(1)
    k = jnp.full((16,), _K, jnp.int32)

    def group_loop(g, carry0):
        off = (gbase + g) * (_CAND_W * 16)
        pltpu.async_copy(ct_hbm.at[pl.ds(off, _CAND_W * 16)], bbuf,
                         sem).wait()

        def bit_loop(i, p):
            b = 30 - i
            cand = p | jnp.broadcast_to(one << b, (16,))

            def cnt_loop(j, cnt):
                x = bbuf[pl.ds(j * 16, 16)]
                return cnt + jnp.where(x >= cand, one, jnp.int32(0))

            cnt = lax.fori_loop(0, _CAND_W, cnt_loop,
                                jnp.zeros((16,), jnp.int32), unroll=16)
            return jnp.where(cnt >= k, cand, p)

        p = lax.fori_loop(0, 31, bit_loop, jnp.zeros((16,), jnp.int32))
        vbuf[pl.ds(g * 16, 16)] = p
        return carry0

    lax.fori_loop(0, _RPW // 16, group_loop, 0)
    pltpu.sync_copy(vbuf, v_hbm.at[pl.ds(wid * _RPW, _RPW)])


def _mask_body(a_ref, n_ref, vk_ref, o_ref):
    vk = vk_ref[0]
    p = jnp.transpose(vk, (1, 0))
    adj = jnp.maximum(a_ref[...], 0.0)
    s = adj + n_ref[...]
    o_ref[...] = jnp.where(s >= p, adj, 0.0)


def kernel(A, noise, idx):
    del idx
    n_rows, n_cols = A.shape

    cand = pl.pallas_call(
        _cand_body,
        grid=(pl.cdiv(n_rows, _BLOCK_R),),
        in_specs=[
            pl.BlockSpec((_BLOCK_R, n_cols), lambda i: (i, 0)),
            pl.BlockSpec((_BLOCK_R, n_cols), lambda i: (i, 0)),
        ],
        out_specs=pl.BlockSpec((_BLOCK_R, _CAND_W), lambda i: (i, 0)),
        out_shape=jax.ShapeDtypeStruct((_R_PAD, _CAND_W), jnp.int32),
    )(A, noise)

    ct = jnp.transpose(cand.reshape(_R_PAD // 16, 16, _CAND_W),
                       (0, 2, 1)).reshape(-1)

    mesh = plsc.VectorSubcoreMesh(core_axis_name="c", subcore_axis_name="s")
    sc_select = functools.partial(
        pl.kernel,
        mesh=mesh,
        out_type=jax.ShapeDtypeStruct((_R_PAD,), jnp.int32),
        scratch_types=[
            pltpu.VMEM((_CAND_W * 16,), jnp.int32),
            pltpu.VMEM((_RPW,), jnp.int32),
            pltpu.SemaphoreType.DMA,
        ],
    )(_sc_select_body)
    vk = sc_select(ct)

    vk_f = jax.lax.bitcast_convert_type(vk, jnp.float32)
    vk3 = vk_f.reshape(_R_PAD // _BLOCK_R, 1, _BLOCK_R)

    out = pl.pallas_call(
        _mask_body,
        grid=(pl.cdiv(n_rows, _BLOCK_R),),
        in_specs=[
            pl.BlockSpec((_BLOCK_R, n_cols), lambda i: (i, 0)),
            pl.BlockSpec((_BLOCK_R, n_cols), lambda i: (i, 0)),
            pl.BlockSpec((1, 1, _BLOCK_R), lambda i: (i, 0, 0)),
        ],
        out_specs=pl.BlockSpec((_BLOCK_R, n_cols), lambda i: (i, 0)),
        out_shape=jax.ShapeDtypeStruct((n_rows, n_cols), A.dtype),
    )(A, noise, vk3)
    return out


# speedup vs baseline: 2.6069x; 2.6069x over previous
import functools

import jax
import jax.numpy as jnp
from jax.experimental import pallas as pl
from jax.experimental.pallas import tpu as pltpu

_K = 32
_BLOCK_R = 128
_TOP = 5


def _topk_mask_body(a_ref, n_ref, o_ref, c_ref, *, k):
    rows = a_ref.shape[0]
    cols = a_ref.shape[1]
    full = cols // 128
    tail = cols - full * 128

    neg1 = jnp.int32(-1)

    sr = 32
    for strip in range(rows // sr):
        r0 = strip * sr

        def step(j, ms):
            a = a_ref[r0:r0 + sr, pl.ds(j * 128, 128)]
            n = n_ref[r0:r0 + sr, pl.ds(j * 128, 128)]
            x = jax.lax.bitcast_convert_type(
                jnp.maximum(a, 0.0) + n, jnp.int32)
            out = []
            for m in ms:
                t = jnp.maximum(m, x)
                x = jnp.minimum(m, x)
                out.append(t)
            return tuple(out)

        init = tuple(jnp.full((sr, 128), neg1) for _ in range(_TOP))
        ms = jax.lax.fori_loop(0, full, step, init, unroll=26)
        for i, m in enumerate(ms):
            c_ref[r0:r0 + sr, i * 128:(i + 1) * 128] = m
        at = a_ref[r0:r0 + sr, full * 128:cols]
        nt = n_ref[r0:r0 + sr, full * 128:cols]
        vt = jax.lax.bitcast_convert_type(jnp.maximum(at, 0.0) + nt,
                                          jnp.int32)
        c_ref[r0:r0 + sr, _TOP * 128:_TOP * 128 + tail] = vt

    cand_all = c_ref[...]

    def bit_step(i, p):
        b = 30 - i
        cand = p | jnp.left_shift(jnp.int32(1), b)
        cnt = jnp.sum((cand_all >= cand).astype(jnp.int32), axis=1,
                      keepdims=True)
        return jnp.where(cnt >= k, cand, p)

    p = jax.lax.fori_loop(0, 31, bit_step,
                          jnp.zeros((rows, 1), jnp.int32), unroll=16)

    adj = jnp.maximum(a_ref[...], 0.0)
    v = jax.lax.bitcast_convert_type(adj + n_ref[...], jnp.int32)
    o_ref[...] = jnp.where(v >= p, adj, 0.0)


def kernel(A, noise, idx):
    del idx
    n_rows, n_cols = A.shape
    grid = (pl.cdiv(n_rows, _BLOCK_R),)
    cand_w = _TOP * 128 + (n_cols - (n_cols // 128) * 128)
    out = pl.pallas_call(
        functools.partial(_topk_mask_body, k=_K),
        grid=grid,
        in_specs=[
            pl.BlockSpec((_BLOCK_R, n_cols), lambda i: (i, 0)),
            pl.BlockSpec((_BLOCK_R, n_cols), lambda i: (i, 0)),
        ],
        out_specs=pl.BlockSpec((_BLOCK_R, n_cols), lambda i: (i, 0)),
        out_shape=jax.ShapeDtypeStruct((n_rows, n_cols), A.dtype),
        scratch_shapes=[pltpu.VMEM((_BLOCK_R, cand_w), jnp.int32)],
    )(A, noise)
    return out
